# baseline (speedup 1.0000x reference)
import functools

import jax
import jax.numpy as jnp
from jax import lax
from jax.experimental import pallas as pl
from jax.experimental.pallas import tpu as pltpu
from jax.experimental.pallas import tpu_sc as plsc

_K = 3
_N_X = 4096
_N_Y = 16384
_D_X = 256
_D_Y = 64
_IN_DIM = _D_X + _D_Y
_OUT_DIM = 256
_BY = 512
_BIG = 1e30

_NW = 32
_PTS_PER_W = _N_Y // _NW
_CHUNK = 32
_NCHUNK = _PTS_PER_W // _CHUNK
_ROWS = _CHUNK * _K
_LANES = 16


def _topk_body(pos_y_ref, pos_xT_ref, idx_ref, w_ref):
    py = pos_y_ref[...]
    pxT = pos_xT_ref[...]
    py0 = py[:, 0:1]
    py1 = py[:, 1:2]
    py2 = py[:, 2:3]
    px0 = pxT[0:1, :]
    px1 = pxT[1:2, :]
    px2 = pxT[2:3, :]

    e0 = py0 - px0
    e1 = py1 - px1
    e2 = py2 - px2
    d2e = e0 * e0 + e1 * e1 + e2 * e2

    sq_y = py0 * py0 + py1 * py1 + py2 * py2
    sq_x = px0 * px0 + px1 * px1 + px2 * px2
    dot = lax.dot_general(py, pxT, (((1,), (0,)), ((), ())),
                          precision=lax.Precision.DEFAULT,
                          preferred_element_type=jnp.float32)
    d2d = (sq_y + sq_x) - 2.0 * dot

    iota = lax.broadcasted_iota(jnp.int32, d2d.shape, 1)
    d = d2d
    idxs = []
    wts = []
    for _ in range(_K):
        mk = jnp.min(d, axis=1, keepdims=True)
        ik = jnp.min(jnp.where(d == mk, iota, jnp.int32(_N_X)), axis=1,
                     keepdims=True)
        eqk = iota == ik
        d2ek = jnp.sum(jnp.where(eqk, d2e, jnp.float32(0.0)), axis=1,
                       keepdims=True)
        d = jnp.where(eqk, _BIG, d)
        idxs.append(ik)
        wts.append(1.0 / jnp.maximum(d2ek, jnp.float32(1e-16)))

    den = wts[0] + wts[1] + wts[2]
    idx_ref[...] = jnp.concatenate(idxs, axis=1)
    w_ref[...] = jnp.concatenate(
        [jnp.broadcast_to(w / den, (w.shape[0], _LANES)) for w in wts],
        axis=1)


def _sc_gather_body(x_hbm, idx_hbm, w_hbm, out_hbm, idx_v, w_v, rows_v,
                    out_v, sem):
    wid = lax.axis_index("s") * 2 + lax.axis_index("c")

    pltpu.sync_copy(idx_hbm.at[wid], idx_v)
    pltpu.sync_copy(w_hbm.at[wid], w_v)

    for j in range(_NCHUNK):
        pltpu.async_copy(x_hbm.at[idx_v.at[j]], rows_v, sem).wait()

        def point(i, _):
            r = i * _K
            for c in range(_D_X // _LANES):
                acc = None
                for k in range(_K):
                    wv = w_v[j * _CHUNK + i, pl.ds(k * _LANES, _LANES)]
                    chunk = rows_v[r + k, pl.ds(c * _LANES, _LANES)]
                    acc = wv * chunk if acc is None else acc + wv * chunk
                out_v[i, pl.ds(c * _LANES, _LANES)] = acc
            return 0

        lax.fori_loop(0, _CHUNK, point, 0)
        pltpu.sync_copy(out_v, out_hbm.at[wid, j])


def _tail_body(interp_ref, y_ref, W_ref, gamma_ref, beta_ref, out_ref):
    Wm = W_ref[...]
    h = jnp.dot(interp_ref[...], Wm[:_D_X, :],
                preferred_element_type=jnp.float32)
    h = h + jnp.dot(y_ref[...], Wm[_D_X:, :],
                    preferred_element_type=jnp.float32)
    mu = jnp.mean(h, axis=-1, keepdims=True)
    var = jnp.mean((h - mu) ** 2, axis=-1, keepdims=True)
    hn = (h - mu) / jnp.sqrt(var + jnp.float32(1e-5))
    hn = hn * gamma_ref[...] + beta_ref[...]
    out_ref[...] = jnp.maximum(hn, jnp.float32(0.0))


def kernel(pos_x_origin, x, batch_x, pos_y_original, y, batch_y, W, gamma,
           beta):
    del batch_x, batch_y
    pos_xT = pos_x_origin.T
    gamma2 = gamma.reshape(1, _OUT_DIM)
    beta2 = beta.reshape(1, _OUT_DIM)
    grid = (_N_Y // _BY,)

    idx, wexp = pl.pallas_call(
        _topk_body,
        grid=grid,
        in_specs=[
            pl.BlockSpec((_BY, 3), lambda i: (i, 0)),
            pl.BlockSpec((3, _N_X), lambda i: (0, 0)),
        ],
        out_specs=[
            pl.BlockSpec((_BY, _K), lambda i: (i, 0)),
            pl.BlockSpec((_BY, _K * _LANES), lambda i: (i, 0)),
        ],
        out_shape=[
            jax.ShapeDtypeStruct((_N_Y, _K), jnp.int32),
            jax.ShapeDtypeStruct((_N_Y, _K * _LANES), jnp.float32),
        ],
    )(pos_y_original, pos_xT)

    idx_sc = idx.reshape(_NW, _NCHUNK, _ROWS)
    w_sc = wexp.reshape(_NW, _PTS_PER_W, _K * _LANES)
    mesh = plsc.VectorSubcoreMesh(core_axis_name="c", subcore_axis_name="s")
    interp = pl.kernel(
        _sc_gather_body,
        mesh=mesh,
        out_type=jax.ShapeDtypeStruct((_NW, _NCHUNK, _CHUNK, _D_X),
                                      jnp.float32),
        scratch_types=[
            pltpu.VMEM((_NCHUNK, _ROWS), jnp.int32),
            pltpu.VMEM((_PTS_PER_W, _K * _LANES), jnp.float32),
            pltpu.VMEM((_ROWS, _D_X), jnp.float32),
            pltpu.VMEM((_CHUNK, _D_X), jnp.float32),
            pltpu.SemaphoreType.DMA,
        ],
    )(x, idx_sc, w_sc)
    interp = interp.reshape(_N_Y, _D_X)

    return pl.pallas_call(
        _tail_body,
        grid=grid,
        in_specs=[
            pl.BlockSpec((_BY, _D_X), lambda i: (i, 0)),
            pl.BlockSpec((_BY, _D_Y), lambda i: (i, 0)),
            pl.BlockSpec((_IN_DIM, _OUT_DIM), lambda i: (0, 0)),
            pl.BlockSpec((1, _OUT_DIM), lambda i: (0, 0)),
            pl.BlockSpec((1, _OUT_DIM), lambda i: (0, 0)),
        ],
        out_specs=pl.BlockSpec((_BY, _OUT_DIM), lambda i: (i, 0)),
        out_shape=jax.ShapeDtypeStruct((_N_Y, _OUT_DIM), jnp.float32),
    )(interp, y, W, gamma2, beta2)


# speedup vs baseline: 1.4430x; 1.4430x over previous
import jax
import jax.numpy as jnp
from jax import lax
from jax.experimental import pallas as pl
from jax.experimental.pallas import tpu as pltpu
from jax.experimental.pallas import tpu_sc as plsc

_K = 3
_N_X = 4096
_N_Y = 16384
_D_X = 256
_D_Y = 64
_IN_DIM = _D_X + _D_Y
_OUT_DIM = 256
_BY = 512
_BIG = 1e30

_NW = 32
_PTS_PER_W = _N_Y // _NW
_CHUNK = 32
_NCHUNK = _PTS_PER_W // _CHUNK
_ROWS = _CHUNK * _K
_LANES = 16
_NGRP = _PTS_PER_W // _LANES


def _topk_body(pos_y_ref, pos_xT_ref, idx_ref):
    py = pos_y_ref[...]
    pxT = pos_xT_ref[...]
    py0 = py[:, 0:1]
    py1 = py[:, 1:2]
    py2 = py[:, 2:3]
    px0 = pxT[0:1, :]
    px1 = pxT[1:2, :]
    px2 = pxT[2:3, :]

    sq_y = py0 * py0 + py1 * py1 + py2 * py2
    sq_x = px0 * px0 + px1 * px1 + px2 * px2
    dot = lax.dot_general(py, pxT, (((1,), (0,)), ((), ())),
                          precision=lax.Precision.DEFAULT,
                          preferred_element_type=jnp.float32)
    d = (sq_y + sq_x) - 2.0 * dot

    iota = lax.broadcasted_iota(jnp.int32, d.shape, 1)
    idxs = []
    for k in range(_K):
        mk = jnp.min(d, axis=1, keepdims=True)
        ik = jnp.min(jnp.where(d == mk, iota, jnp.int32(_N_X)), axis=1,
                     keepdims=True)
        idxs.append(ik)
        if k + 1 < _K:
            d = jnp.where(iota == ik, _BIG, d)
    idx_ref[...] = jnp.concatenate(idxs, axis=1)


def _sc_body(x_hbm, idx_hbm, idxf_hbm, px0_hbm, px1_hbm, px2_hbm,
             py0_hbm, py1_hbm, py2_hbm, out_hbm,
             idx_v, idxf_v, wn_v, px0_v, px1_v, px2_v, py0_v, py1_v, py2_v,
             rows0_v, rows1_v, out_v, sem0, sem1):
    wid = lax.axis_index("s") * 2 + lax.axis_index("c")
    base = wid * _PTS_PER_W

    pltpu.sync_copy(idx_hbm.at[wid], idx_v)
    pltpu.sync_copy(idxf_hbm.at[wid], idxf_v)
    pltpu.sync_copy(px0_hbm, px0_v)
    pltpu.sync_copy(px1_hbm, px1_v)
    pltpu.sync_copy(px2_hbm, px2_v)
    pltpu.sync_copy(py0_hbm.at[pl.ds(base, _PTS_PER_W)], py0_v)
    pltpu.sync_copy(py1_hbm.at[pl.ds(base, _PTS_PER_W)], py1_v)
    pltpu.sync_copy(py2_hbm.at[pl.ds(base, _PTS_PER_W)], py2_v)

    iota3 = lax.iota(jnp.int32, _LANES) * 3

    for g in range(_NGRP):
        ws = []
        for k in range(_K):
            sel = iota3 + (g * (_LANES * _K) + k)
            ik = plsc.load_gather(idxf_v, [sel])
            gx0 = plsc.load_gather(px0_v, [ik])
            gx1 = plsc.load_gather(px1_v, [ik])
            gx2 = plsc.load_gather(px2_v, [ik])
            e0 = py0_v[pl.ds(g * _LANES, _LANES)] - gx0
            e1 = py1_v[pl.ds(g * _LANES, _LANES)] - gx1
            e2 = py2_v[pl.ds(g * _LANES, _LANES)] - gx2
            d2 = e0 * e0 + e1 * e1 + e2 * e2
            ws.append(1.0 / jnp.maximum(d2, jnp.float32(1e-16)))
        inv_den = 1.0 / (ws[0] + ws[1] + ws[2])
        woff = g * (_LANES * _K)
        for k in range(_K):
            plsc.store_scatter(wn_v, [iota3 + (woff + k)], ws[k] * inv_den)

    rows = [rows0_v, rows1_v]
    sems = [sem0, sem1]
    copies = [None, None]
    copies[0] = pltpu.async_copy(x_hbm.at[idx_v.at[0]], rows0_v, sem0)
    for j in range(_NCHUNK):
        b = j % 2
        copies[b].wait()
        if j + 1 < _NCHUNK:
            nb = (j + 1) % 2
            copies[nb] = pltpu.async_copy(x_hbm.at[idx_v.at[j + 1]],
                                          rows[nb], sems[nb])

        def point(i, _):
            p3 = (j * _CHUNK + i) * _K
            r = i * _K
            for c in range(_D_X // _LANES):
                acc = None
                for k in range(_K):
                    wv = plsc.load_gather(
                        wn_v, [jnp.broadcast_to(p3 + k, (_LANES,))])
                    chunk = rows[b][r + k, pl.ds(c * _LANES, _LANES)]
                    acc = wv * chunk if acc is None else acc + wv * chunk
                out_v[i, pl.ds(c * _LANES, _LANES)] = acc
            return 0

        lax.fori_loop(0, _CHUNK, point, 0)
        pltpu.sync_copy(out_v, out_hbm.at[wid, j])


def _tail_body(interp_ref, y_ref, W_ref, gamma_ref, beta_ref, out_ref):
    Wm = W_ref[...]
    h = jnp.dot(interp_ref[...], Wm[:_D_X, :],
                preferred_element_type=jnp.float32)
    h = h + jnp.dot(y_ref[...], Wm[_D_X:, :],
                    preferred_element_type=jnp.float32)
    mu = jnp.mean(h, axis=-1, keepdims=True)
    var = jnp.mean((h - mu) ** 2, axis=-1, keepdims=True)
    hn = (h - mu) / jnp.sqrt(var + jnp.float32(1e-5))
    hn = hn * gamma_ref[...] + beta_ref[...]
    out_ref[...] = jnp.maximum(hn, jnp.float32(0.0))


def kernel(pos_x_origin, x, batch_x, pos_y_original, y, batch_y, W, gamma,
           beta):
    del batch_x, batch_y
    pos_xT = pos_x_origin.T
    pos_yT = pos_y_original.T
    gamma2 = gamma.reshape(1, _OUT_DIM)
    beta2 = beta.reshape(1, _OUT_DIM)
    grid = (_N_Y // _BY,)

    idx = pl.pallas_call(
        _topk_body,
        grid=grid,
        in_specs=[
            pl.BlockSpec((_BY, 3), lambda i: (i, 0)),
            pl.BlockSpec((3, _N_X), lambda i: (0, 0)),
        ],
        out_specs=pl.BlockSpec((_BY, _K), lambda i: (i, 0)),
        out_shape=jax.ShapeDtypeStruct((_N_Y, _K), jnp.int32),
    )(pos_y_original, pos_xT)

    idx_sc = idx.reshape(_NW, _NCHUNK, _ROWS)
    mesh = plsc.VectorSubcoreMesh(core_axis_name="c", subcore_axis_name="s")
    interp = pl.kernel(
        _sc_body,
        mesh=mesh,
        compiler_params=pltpu.CompilerParams(needs_layout_passes=False),
        out_type=jax.ShapeDtypeStruct((_NW, _NCHUNK, _CHUNK, _D_X),
                                      jnp.float32),
        scratch_types=[
            pltpu.VMEM((_NCHUNK, _ROWS), jnp.int32),
            pltpu.VMEM((_PTS_PER_W * _K,), jnp.int32),
            pltpu.VMEM((_PTS_PER_W * _K,), jnp.float32),
            pltpu.VMEM((_N_X,), jnp.float32),
            pltpu.VMEM((_N_X,), jnp.float32),
            pltpu.VMEM((_N_X,), jnp.float32),
            pltpu.VMEM((_PTS_PER_W,), jnp.float32),
            pltpu.VMEM((_PTS_PER_W,), jnp.float32),
            pltpu.VMEM((_PTS_PER_W,), jnp.float32),
            pltpu.VMEM((_ROWS, _D_X), jnp.float32),
            pltpu.VMEM((_ROWS, _D_X), jnp.float32),
            pltpu.VMEM((_CHUNK, _D_X), jnp.float32),
            pltpu.SemaphoreType.DMA,
            pltpu.SemaphoreType.DMA,
        ],
    )(x, idx_sc, idx.reshape(_NW, _PTS_PER_W * _K),
      pos_xT[0], pos_xT[1], pos_xT[2],
      pos_yT[0], pos_yT[1], pos_yT[2])
    interp = interp.reshape(_N_Y, _D_X)

    return pl.pallas_call(
        _tail_body,
        grid=grid,
        in_specs=[
            pl.BlockSpec((_BY, _D_X), lambda i: (i, 0)),
            pl.BlockSpec((_BY, _D_Y), lambda i: (i, 0)),
            pl.BlockSpec((_IN_DIM, _OUT_DIM), lambda i: (0, 0)),
            pl.BlockSpec((1, _OUT_DIM), lambda i: (0, 0)),
            pl.BlockSpec((1, _OUT_DIM), lambda i: (0, 0)),
        ],
        out_specs=pl.BlockSpec((_BY, _OUT_DIM), lambda i: (i, 0)),
        out_shape=jax.ShapeDtypeStruct((_N_Y, _OUT_DIM), jnp.float32),
    )(interp, y, W, gamma2, beta2)


# speedup vs baseline: 1.5392x; 1.0666x over previous
import jax
import jax.numpy as jnp
from jax import lax
from jax.experimental import pallas as pl
from jax.experimental.pallas import tpu as pltpu
from jax.experimental.pallas import tpu_sc as plsc

_K = 3
_N_X = 4096
_N_Y = 16384
_D_X = 256
_D_Y = 64
_IN_DIM = _D_X + _D_Y
_OUT_DIM = 256
_BY = 512
_BIG = 1e30
_NSPLIT = 2

_NW = 32
_CHUNK = 32
_ROWS = _CHUNK * _K
_LANES = 16


def _topk_body(pos_y_ref, pos_xT_ref, idx_ref):
    py = pos_y_ref[...]
    pxT = pos_xT_ref[...]
    py0 = py[:, 0:1]
    py1 = py[:, 1:2]
    py2 = py[:, 2:3]
    px0 = pxT[0:1, :]
    px1 = pxT[1:2, :]
    px2 = pxT[2:3, :]

    sq_y = py0 * py0 + py1 * py1 + py2 * py2
    sq_x = px0 * px0 + px1 * px1 + px2 * px2
    dot = lax.dot_general(py, pxT, (((1,), (0,)), ((), ())),
                          precision=lax.Precision.DEFAULT,
                          preferred_element_type=jnp.float32)
    d = (sq_y + sq_x) - 2.0 * dot

    iota = lax.broadcasted_iota(jnp.int32, d.shape, 1)
    idxs = []
    for k in range(_K):
        mk = jnp.min(d, axis=1, keepdims=True)
        ik = jnp.min(jnp.where(d == mk, iota, jnp.int32(_N_X)), axis=1,
                     keepdims=True)
        idxs.append(ik)
        if k + 1 < _K:
            d = jnp.where(iota == ik, _BIG, d)
    idx_ref[...] = jnp.concatenate(idxs, axis=1)


def _make_sc_body(npts, nchunk, ngrp):
    def _sc_body(x_hbm, idx_hbm, idxf_hbm, px0_hbm, px1_hbm, px2_hbm,
                 py0_hbm, py1_hbm, py2_hbm, out_hbm,
                 idx_v, idxf_v, wn_v, px0_v, px1_v, px2_v,
                 py0_v, py1_v, py2_v, rows0_v, rows1_v, out_v, sem0, sem1):
        wid = lax.axis_index("s") * 2 + lax.axis_index("c")
        base = wid * npts

        pltpu.sync_copy(idx_hbm.at[wid], idx_v)
        pltpu.sync_copy(idxf_hbm.at[wid], idxf_v)
        pltpu.sync_copy(px0_hbm, px0_v)
        pltpu.sync_copy(px1_hbm, px1_v)
        pltpu.sync_copy(px2_hbm, px2_v)
        pltpu.sync_copy(py0_hbm.at[pl.ds(base, npts)], py0_v)
        pltpu.sync_copy(py1_hbm.at[pl.ds(base, npts)], py1_v)
        pltpu.sync_copy(py2_hbm.at[pl.ds(base, npts)], py2_v)

        iota3 = lax.iota(jnp.int32, _LANES) * 3

        for g in range(ngrp):
            ws = []
            for k in range(_K):
                sel = iota3 + (g * (_LANES * _K) + k)
                ik = plsc.load_gather(idxf_v, [sel])
                gx0 = plsc.load_gather(px0_v, [ik])
                gx1 = plsc.load_gather(px1_v, [ik])
                gx2 = plsc.load_gather(px2_v, [ik])
                e0 = py0_v[pl.ds(g * _LANES, _LANES)] - gx0
                e1 = py1_v[pl.ds(g * _LANES, _LANES)] - gx1
                e2 = py2_v[pl.ds(g * _LANES, _LANES)] - gx2
                d2 = e0 * e0 + e1 * e1 + e2 * e2
                ws.append(1.0 / jnp.maximum(d2, jnp.float32(1e-16)))
            inv_den = 1.0 / (ws[0] + ws[1] + ws[2])
            woff = g * (_LANES * _K)
            for k in range(_K):
                plsc.store_scatter(wn_v, [iota3 + (woff + k)],
                                   ws[k] * inv_den)

        rows = [rows0_v, rows1_v]
        sems = [sem0, sem1]
        copies = [None, None]
        copies[0] = pltpu.async_copy(x_hbm.at[idx_v.at[0]], rows0_v, sem0)
        for j in range(nchunk):
            b = j % 2
            copies[b].wait()
            if j + 1 < nchunk:
                nb = (j + 1) % 2
                copies[nb] = pltpu.async_copy(x_hbm.at[idx_v.at[j + 1]],
                                              rows[nb], sems[nb])

            def point(i, _):
                p3 = (j * _CHUNK + i) * _K
                r = i * _K
                for c in range(_D_X // _LANES):
                    acc = None
                    for k in range(_K):
                        wv = plsc.load_gather(
                            wn_v, [jnp.broadcast_to(p3 + k, (_LANES,))])
                        chunk = rows[b][r + k, pl.ds(c * _LANES, _LANES)]
                        acc = wv * chunk if acc is None else acc + wv * chunk
                    out_v[i, pl.ds(c * _LANES, _LANES)] = acc
                return 0

            lax.fori_loop(0, _CHUNK, point, 0)
            pltpu.sync_copy(out_v, out_hbm.at[wid, j])

    return _sc_body


def _sc_stage(x, idx, px, py, n_y):
    npts = n_y // _NW
    nchunk = npts // _CHUNK
    ngrp = npts // _LANES
    idx_sc = idx.reshape(_NW, nchunk, _ROWS)
    idx_flat = idx.reshape(_NW, npts * _K)
    mesh = plsc.VectorSubcoreMesh(core_axis_name="c", subcore_axis_name="s")
    interp = pl.kernel(
        _make_sc_body(npts, nchunk, ngrp),
        mesh=mesh,
        compiler_params=pltpu.CompilerParams(needs_layout_passes=False),
        out_type=jax.ShapeDtypeStruct((_NW, nchunk, _CHUNK, _D_X),
                                      jnp.float32),
        scratch_types=[
            pltpu.VMEM((nchunk, _ROWS), jnp.int32),
            pltpu.VMEM((npts * _K,), jnp.int32),
            pltpu.VMEM((npts * _K,), jnp.float32),
            pltpu.VMEM((_N_X,), jnp.float32),
            pltpu.VMEM((_N_X,), jnp.float32),
            pltpu.VMEM((_N_X,), jnp.float32),
            pltpu.VMEM((npts,), jnp.float32),
            pltpu.VMEM((npts,), jnp.float32),
            pltpu.VMEM((npts,), jnp.float32),
            pltpu.VMEM((_ROWS, _D_X), jnp.float32),
            pltpu.VMEM((_ROWS, _D_X), jnp.float32),
            pltpu.VMEM((_CHUNK, _D_X), jnp.float32),
            pltpu.SemaphoreType.DMA,
            pltpu.SemaphoreType.DMA,
        ],
    )(x, idx_sc, idx_flat, px[0], px[1], px[2], py[0], py[1], py[2])
    return interp.reshape(n_y, _D_X)


def _tail_body(interp_ref, y_ref, W_ref, gamma_ref, beta_ref, out_ref):
    Wm = W_ref[...]
    h = jnp.dot(interp_ref[...], Wm[:_D_X, :],
                preferred_element_type=jnp.float32)
    h = h + jnp.dot(y_ref[...], Wm[_D_X:, :],
                    preferred_element_type=jnp.float32)
    mu = jnp.mean(h, axis=-1, keepdims=True)
    var = jnp.mean((h - mu) ** 2, axis=-1, keepdims=True)
    hn = (h - mu) / jnp.sqrt(var + jnp.float32(1e-5))
    hn = hn * gamma_ref[...] + beta_ref[...]
    out_ref[...] = jnp.maximum(hn, jnp.float32(0.0))


def kernel(pos_x_origin, x, batch_x, pos_y_original, y, batch_y, W, gamma,
           beta):
    del batch_x, batch_y
    pos_xT = pos_x_origin.T
    gamma2 = gamma.reshape(1, _OUT_DIM)
    beta2 = beta.reshape(1, _OUT_DIM)
    px = (pos_xT[0], pos_xT[1], pos_xT[2])

    n_h = _N_Y // _NSPLIT
    grid = (n_h // _BY,)
    outs = []
    for h in range(_NSPLIT):
        sl = slice(h * n_h, (h + 1) * n_h)
        pos_y_h = pos_y_original[sl]
        pos_yT_h = pos_y_h.T
        py = (pos_yT_h[0], pos_yT_h[1], pos_yT_h[2])

        idx = pl.pallas_call(
            _topk_body,
            grid=grid,
            in_specs=[
                pl.BlockSpec((_BY, 3), lambda i: (i, 0)),
                pl.BlockSpec((3, _N_X), lambda i: (0, 0)),
            ],
            out_specs=pl.BlockSpec((_BY, _K), lambda i: (i, 0)),
            out_shape=jax.ShapeDtypeStruct((n_h, _K), jnp.int32),
        )(pos_y_h, pos_xT)

        interp = _sc_stage(x, idx, px, py, n_h)

        outs.append(pl.pallas_call(
            _tail_body,
            grid=grid,
            in_specs=[
                pl.BlockSpec((_BY, _D_X), lambda i: (i, 0)),
                pl.BlockSpec((_BY, _D_Y), lambda i: (i, 0)),
                pl.BlockSpec((_IN_DIM, _OUT_DIM), lambda i: (0, 0)),
                pl.BlockSpec((1, _OUT_DIM), lambda i: (0, 0)),
                pl.BlockSpec((1, _OUT_DIM), lambda i: (0, 0)),
            ],
            out_specs=pl.BlockSpec((_BY, _OUT_DIM), lambda i: (i, 0)),
            out_shape=jax.ShapeDtypeStruct((n_h, _OUT_DIM), jnp.float32),
        )(interp, y[sl], W, gamma2, beta2))

    return jnp.concatenate(outs, axis=0)


# speedup vs baseline: 1.5550x; 1.0103x over previous
import jax
import jax.numpy as jnp
from jax import lax
from jax.experimental import pallas as pl
from jax.experimental.pallas import tpu as pltpu
from jax.experimental.pallas import tpu_sc as plsc

_K = 3
_N_X = 4096
_N_Y = 16384
_D_X = 256
_D_Y = 64
_IN_DIM = _D_X + _D_Y
_OUT_DIM = 256
_BY = 512
_BIG = 1e30
_NSPLIT = 2

_NW = 32
_CHUNK = 32
_ROWS = _CHUNK * _K
_LANES = 16


def _topk_body(pos_y_ref, pos_xT_ref, idx_ref):
    py = pos_y_ref[...]
    pxT = pos_xT_ref[...]
    py0 = py[:, 0:1]
    py1 = py[:, 1:2]
    py2 = py[:, 2:3]
    px0 = pxT[0:1, :]
    px1 = pxT[1:2, :]
    px2 = pxT[2:3, :]

    sq_y = py0 * py0 + py1 * py1 + py2 * py2
    sq_x = px0 * px0 + px1 * px1 + px2 * px2
    dot = lax.dot_general(py, pxT, (((1,), (0,)), ((), ())),
                          precision=lax.Precision.DEFAULT,
                          preferred_element_type=jnp.float32)
    d = (sq_y + sq_x) - 2.0 * dot

    iota = lax.broadcasted_iota(jnp.int32, d.shape, 1)
    idxs = []
    for k in range(_K):
        mk = jnp.min(d, axis=1, keepdims=True)
        ik = jnp.min(jnp.where(d == mk, iota, jnp.int32(_N_X)), axis=1,
                     keepdims=True)
        idxs.append(ik)
        if k + 1 < _K:
            d = jnp.where(iota == ik, _BIG, d)
    idx_ref[...] = jnp.concatenate(idxs, axis=1)


def _make_sc_body(npts, nchunk, ngrp):
    def _sc_body(x_hbm, idx_hbm, idxf_hbm, px0_hbm, px1_hbm, px2_hbm,
                 py0_hbm, py1_hbm, py2_hbm, out_hbm,
                 idx_v, idxf_v, wn_v, px0_v, px1_v, px2_v,
                 py0_v, py1_v, py2_v, rows0_v, rows1_v, out_v, sem0, sem1):
        wid = lax.axis_index("s") * 2 + lax.axis_index("c")
        base = wid * npts

        pltpu.sync_copy(idx_hbm.at[wid], idx_v)
        pltpu.sync_copy(idxf_hbm.at[wid], idxf_v)
        pltpu.sync_copy(px0_hbm, px0_v)
        pltpu.sync_copy(px1_hbm, px1_v)
        pltpu.sync_copy(px2_hbm, px2_v)
        pltpu.sync_copy(py0_hbm.at[pl.ds(base, npts)], py0_v)
        pltpu.sync_copy(py1_hbm.at[pl.ds(base, npts)], py1_v)
        pltpu.sync_copy(py2_hbm.at[pl.ds(base, npts)], py2_v)

        iota3 = lax.iota(jnp.int32, _LANES) * 3

        for g in range(ngrp):
            ws = []
            for k in range(_K):
                sel = iota3 + (g * (_LANES * _K) + k)
                ik = plsc.load_gather(idxf_v, [sel])
                gx0 = plsc.load_gather(px0_v, [ik])
                gx1 = plsc.load_gather(px1_v, [ik])
                gx2 = plsc.load_gather(px2_v, [ik])
                e0 = py0_v[pl.ds(g * _LANES, _LANES)] - gx0
                e1 = py1_v[pl.ds(g * _LANES, _LANES)] - gx1
                e2 = py2_v[pl.ds(g * _LANES, _LANES)] - gx2
                d2 = e0 * e0 + e1 * e1 + e2 * e2
                ws.append(1.0 / jnp.maximum(d2, jnp.float32(1e-16)))
            inv_den = 1.0 / (ws[0] + ws[1] + ws[2])
            woff = g * (_LANES * _K)
            for k in range(_K):
                plsc.store_scatter(wn_v, [iota3 + (woff + k)],
                                   ws[k] * inv_den)

        rows = [rows0_v, rows1_v]
        sems = [sem0, sem1]
        copies = [None, None]
        copies[0] = pltpu.async_copy(x_hbm.at[idx_v.at[0]], rows0_v, sem0)
        for j in range(nchunk):
            b = j % 2
            copies[b].wait()
            if j + 1 < nchunk:
                nb = (j + 1) % 2
                copies[nb] = pltpu.async_copy(x_hbm.at[idx_v.at[j + 1]],
                                              rows[nb], sems[nb])

            def point(i, _):
                p3 = (j * _CHUNK + i) * _K
                r = i * _K
                wvs = [plsc.load_gather(
                           wn_v, [jnp.broadcast_to(p3 + k, (_LANES,))])
                       for k in range(_K)]
                for c in range(_D_X // _LANES):
                    acc = None
                    for k in range(_K):
                        chunk = rows[b][r + k, pl.ds(c * _LANES, _LANES)]
                        acc = (wvs[k] * chunk if acc is None
                               else acc + wvs[k] * chunk)
                    out_v[i, pl.ds(c * _LANES, _LANES)] = acc
                return 0

            lax.fori_loop(0, _CHUNK, point, 0)
            pltpu.sync_copy(out_v, out_hbm.at[wid, j])

    return _sc_body


def _sc_stage(x, idx, px, py, n_y):
    npts = n_y // _NW
    nchunk = npts // _CHUNK
    ngrp = npts // _LANES
    idx_sc = idx.reshape(_NW, nchunk, _ROWS)
    idx_flat = idx.reshape(_NW, npts * _K)
    mesh = plsc.VectorSubcoreMesh(core_axis_name="c", subcore_axis_name="s")
    interp = pl.kernel(
        _make_sc_body(npts, nchunk, ngrp),
        mesh=mesh,
        compiler_params=pltpu.CompilerParams(needs_layout_passes=False),
        out_type=jax.ShapeDtypeStruct((_NW, nchunk, _CHUNK, _D_X),
                                      jnp.float32),
        scratch_types=[
            pltpu.VMEM((nchunk, _ROWS), jnp.int32),
            pltpu.VMEM((npts * _K,), jnp.int32),
            pltpu.VMEM((npts * _K,), jnp.float32),
            pltpu.VMEM((_N_X,), jnp.float32),
            pltpu.VMEM((_N_X,), jnp.float32),
            pltpu.VMEM((_N_X,), jnp.float32),
            pltpu.VMEM((npts,), jnp.float32),
            pltpu.VMEM((npts,), jnp.float32),
            pltpu.VMEM((npts,), jnp.float32),
            pltpu.VMEM((_ROWS, _D_X), jnp.float32),
            pltpu.VMEM((_ROWS, _D_X), jnp.float32),
            pltpu.VMEM((_CHUNK, _D_X), jnp.float32),
            pltpu.SemaphoreType.DMA,
            pltpu.SemaphoreType.DMA,
        ],
    )(x, idx_sc, idx_flat, px[0], px[1], px[2], py[0], py[1], py[2])
    return interp.reshape(n_y, _D_X)


def _tail_body(interp_ref, y_ref, W_ref, gamma_ref, beta_ref, out_ref):
    Wm = W_ref[...]
    h = jnp.dot(interp_ref[...], Wm[:_D_X, :],
                preferred_element_type=jnp.float32)
    h = h + jnp.dot(y_ref[...], Wm[_D_X:, :],
                    preferred_element_type=jnp.float32)
    mu = jnp.mean(h, axis=-1, keepdims=True)
    var = jnp.mean((h - mu) ** 2, axis=-1, keepdims=True)
    hn = (h - mu) / jnp.sqrt(var + jnp.float32(1e-5))
    hn = hn * gamma_ref[...] + beta_ref[...]
    out_ref[...] = jnp.maximum(hn, jnp.float32(0.0))


def kernel(pos_x_origin, x, batch_x, pos_y_original, y, batch_y, W, gamma,
           beta):
    del batch_x, batch_y
    pos_xT = pos_x_origin.T
    gamma2 = gamma.reshape(1, _OUT_DIM)
    beta2 = beta.reshape(1, _OUT_DIM)
    px = (pos_xT[0], pos_xT[1], pos_xT[2])

    n_h = _N_Y // _NSPLIT
    grid = (n_h // _BY,)
    outs = []
    for h in range(_NSPLIT):
        sl = slice(h * n_h, (h + 1) * n_h)
        pos_y_h = pos_y_original[sl]
        pos_yT_h = pos_y_h.T
        py = (pos_yT_h[0], pos_yT_h[1], pos_yT_h[2])

        idx = pl.pallas_call(
            _topk_body,
            grid=grid,
            in_specs=[
                pl.BlockSpec((_BY, 3), lambda i: (i, 0)),
                pl.BlockSpec((3, _N_X), lambda i: (0, 0)),
            ],
            out_specs=pl.BlockSpec((_BY, _K), lambda i: (i, 0)),
            out_shape=jax.ShapeDtypeStruct((n_h, _K), jnp.int32),
        )(pos_y_h, pos_xT)

        interp = _sc_stage(x, idx, px, py, n_h)

        outs.append(pl.pallas_call(
            _tail_body,
            grid=grid,
            in_specs=[
                pl.BlockSpec((_BY, _D_X), lambda i: (i, 0)),
                pl.BlockSpec((_BY, _D_Y), lambda i: (i, 0)),
                pl.BlockSpec((_IN_DIM, _OUT_DIM), lambda i: (0, 0)),
                pl.BlockSpec((1, _OUT_DIM), lambda i: (0, 0)),
                pl.BlockSpec((1, _OUT_DIM), lambda i: (0, 0)),
            ],
            out_specs=pl.BlockSpec((_BY, _OUT_DIM), lambda i: (i, 0)),
            out_shape=jax.ShapeDtypeStruct((n_h, _OUT_DIM), jnp.float32),
        )(interp, y[sl], W, gamma2, beta2))

    return jnp.concatenate(outs, axis=0)
